# baseline (speedup 1.0000x reference)
import functools

import jax
import jax.numpy as jnp
from jax import lax
from jax.experimental import pallas as pl
from jax.experimental.pallas import tpu as pltpu
from jax.experimental.pallas import tpu_sc as plsc

EPS = 1e-12


def kernel(input_ids, word_emb, pos_emb, gamma, beta):
    B, S = input_ids.shape
    V, H = word_emb.shape
    assert H == 64
    NW = 32
    seq_per_w = B // NW

    mesh = plsc.VectorSubcoreMesh(core_axis_name="c", subcore_axis_name="s")

    @functools.partial(
        pl.kernel,
        out_type=jax.ShapeDtypeStruct((B, S, H), jnp.float32),
        mesh=mesh,
        scratch_types=[
            pltpu.VMEM((S,), jnp.int32),
            pltpu.VMEM((S, H), jnp.float32),
            pltpu.VMEM((S, H), jnp.float32),
            pltpu.VMEM((H,), jnp.float32),
            pltpu.VMEM((H,), jnp.float32),
            pltpu.SemaphoreType.DMA,
        ],
        compiler_params=pltpu.CompilerParams(
            needs_layout_passes=False, use_tc_tiling_on_sc=False),
    )
    def emb_ln(ids_hbm, wemb_hbm, pemb_hbm, g_hbm, b_hbm, out_hbm,
               idx_v, rows_v, pos_v, g_v, b_v, sem):
        cid = lax.axis_index("c")
        sid = lax.axis_index("s")
        wid = sid * 2 + cid

        pltpu.sync_copy(pemb_hbm.at[pl.ds(0, S)], pos_v)
        pltpu.sync_copy(g_hbm, g_v)
        pltpu.sync_copy(b_hbm, b_v)
        g = [g_v[pl.ds(16 * j, 16)] for j in range(4)]
        bt = [b_v[pl.ds(16 * j, 16)] for j in range(4)]

        def seq_body(i, carry):
            seq = wid * seq_per_w + i
            pltpu.sync_copy(ids_hbm.at[seq], idx_v)
            cp1 = pltpu.async_copy(
                wemb_hbm.at[idx_v.at[pl.ds(0, 128)]],
                rows_v.at[pl.ds(0, 128)], sem)
            cp2 = pltpu.async_copy(
                wemb_hbm.at[idx_v.at[pl.ds(128, S - 128)]],
                rows_v.at[pl.ds(128, S - 128)], sem)
            cp1.wait()
            cp2.wait()

            def tok_body(t, c):
                x = [rows_v[t, pl.ds(16 * j, 16)] + pos_v[t, pl.ds(16 * j, 16)]
                     for j in range(4)]
                s = (x[0] + x[1]) + (x[2] + x[3])
                mean = jnp.sum(s) * (1.0 / 64.0)
                q = (x[0] * x[0] + x[1] * x[1]) + (x[2] * x[2] + x[3] * x[3])
                var = jnp.sum(q) * (1.0 / 64.0) - mean * mean
                v = jnp.maximum(var, 0.0) + EPS
                iv = lax.bitcast_convert_type(v, jnp.int32)
                y = lax.bitcast_convert_type(
                    jnp.int32(0x5F3759DF) - (iv >> 1), jnp.float32)
                y = y * (1.5 - 0.5 * v * y * y)
                y = y * (1.5 - 0.5 * v * y * y)
                y = y * (1.5 - 0.5 * v * y * y)
                for j in range(4):
                    rows_v[t, pl.ds(16 * j, 16)] = (
                        (x[j] - mean) * y * g[j] + bt[j])
                return c

            lax.fori_loop(0, S, tok_body, 0, unroll=2)
            pltpu.sync_copy(rows_v, out_hbm.at[seq])
            return carry

        lax.fori_loop(0, seq_per_w, seq_body, 0)

    return emb_ln(input_ids, word_emb, pos_emb, gamma, beta)


# speedup vs baseline: 1.8541x; 1.8541x over previous
import functools

import jax
import jax.numpy as jnp
from jax import lax
from jax.experimental import pallas as pl
from jax.experimental.pallas import tpu as pltpu
from jax.experimental.pallas import tpu_sc as plsc

EPS = 1e-12


def kernel(input_ids, word_emb, pos_emb, gamma, beta):
    B, S = input_ids.shape
    V, H = word_emb.shape
    assert H == 64
    NW = 32
    seq_per_w = B // NW
    S0 = 128
    S1 = S - S0

    mesh = plsc.VectorSubcoreMesh(core_axis_name="c", subcore_axis_name="s")

    @functools.partial(
        pl.kernel,
        out_type=jax.ShapeDtypeStruct((B, S, H), jnp.float32),
        mesh=mesh,
        scratch_types=[
            pltpu.VMEM((2, S), jnp.int32),
            pltpu.VMEM((2, S, H), jnp.float32),
            pltpu.VMEM((2, S, H), jnp.float32),
            pltpu.VMEM((S, H), jnp.float32),
            pltpu.VMEM((H,), jnp.float32),
            pltpu.VMEM((H,), jnp.float32),
            pltpu.SemaphoreType.DMA,
            pltpu.SemaphoreType.DMA,
            pltpu.SemaphoreType.DMA,
            pltpu.SemaphoreType.DMA,
        ],
        compiler_params=pltpu.CompilerParams(
            needs_layout_passes=False, use_tc_tiling_on_sc=False),
    )
    def emb_ln(ids_hbm, wemb_hbm, pemb_hbm, g_hbm, b_hbm, out_hbm,
               idx_v, rows_v, outb_v, pos_v, g_v, b_v,
               gsem0, gsem1, osem0, osem1):
        cid = lax.axis_index("c")
        sid = lax.axis_index("s")
        wid = sid * 2 + cid
        seq0 = wid * seq_per_w
        gsem = [gsem0, gsem1]
        osem = [osem0, osem1]

        pltpu.sync_copy(pemb_hbm.at[pl.ds(0, S)], pos_v)
        pltpu.sync_copy(g_hbm, g_v)
        pltpu.sync_copy(b_hbm, b_v)
        g = [g_v[pl.ds(16 * j, 16)] for j in range(4)]
        bt = [b_v[pl.ds(16 * j, 16)] for j in range(4)]
        fifteen = jnp.full((16,), 15, dtype=jnp.int32)

        def start_gather(seq, b):
            pltpu.sync_copy(ids_hbm.at[seq], idx_v.at[b])
            pltpu.async_copy(
                wemb_hbm.at[idx_v.at[b, pl.ds(0, S0)]],
                rows_v.at[b, pl.ds(0, S0)], gsem[b])
            pltpu.async_copy(
                wemb_hbm.at[idx_v.at[b, pl.ds(S0, S1)]],
                rows_v.at[b, pl.ds(S0, S1)], gsem[b])

        def wait_gather(seq, b):
            pltpu.make_async_copy(
                wemb_hbm.at[idx_v.at[b, pl.ds(0, S0)]],
                rows_v.at[b, pl.ds(0, S0)], gsem[b]).wait()
            pltpu.make_async_copy(
                wemb_hbm.at[idx_v.at[b, pl.ds(S0, S1)]],
                rows_v.at[b, pl.ds(S0, S1)], gsem[b]).wait()

        def drain_out(seq, b):
            pltpu.make_async_copy(
                outb_v.at[b], out_hbm.at[seq], osem[b]).wait()

        start_gather(seq0, 0)

        def pair_body(i2, carry):
            for b in range(2):
                i = i2 * 2 + b
                seq = seq0 + i
                wait_gather(seq, b)

                @pl.when(i + 1 < seq_per_w)
                def _():
                    start_gather(seq + 1, 1 - b)

                @pl.when(i >= 2)
                def _():
                    drain_out(seq - 2, b)

                @plsc.parallel_loop(0, S, unroll=4)
                def tok(t):
                    x = [rows_v[b, t, pl.ds(16 * j, 16)]
                         + pos_v[t, pl.ds(16 * j, 16)] for j in range(4)]
                    s = (x[0] + x[1]) + (x[2] + x[3])
                    q = ((x[0] * x[0] + x[1] * x[1])
                         + (x[2] * x[2] + x[3] * x[3]))
                    tot = jnp.take_along_axis(
                        plsc.cumsum(s), fifteen, axis=0,
                        mode="promise_in_bounds")
                    tot2 = jnp.take_along_axis(
                        plsc.cumsum(q), fifteen, axis=0,
                        mode="promise_in_bounds")
                    mean = tot * (1.0 / 64.0)
                    var = tot2 * (1.0 / 64.0) - mean * mean
                    v = jnp.maximum(var, 0.0) + EPS
                    iv = plsc.bitcast(v, jnp.int32)
                    y = plsc.bitcast(jnp.int32(0x5F3759DF) - (iv >> 1),
                                     jnp.float32)
                    h = 0.5 * v
                    y = y * (1.5 - h * y * y)
                    y = y * (1.5 - h * y * y)
                    y = y * (1.5 - h * y * y)
                    for j in range(4):
                        outb_v[b, t, pl.ds(16 * j, 16)] = (
                            (x[j] - mean) * y * g[j] + bt[j])

                pltpu.async_copy(outb_v.at[b], out_hbm.at[seq], osem[b])
            return carry

        lax.fori_loop(0, seq_per_w // 2, pair_body, 0)
        drain_out(seq0 + seq_per_w - 2, 0)
        drain_out(seq0 + seq_per_w - 1, 1)

    return emb_ln(input_ids, word_emb, pos_emb, gamma, beta)


# speedup vs baseline: 2.2365x; 1.2063x over previous
import functools

import jax
import jax.numpy as jnp
from jax import lax
from jax.experimental import pallas as pl
from jax.experimental.pallas import tpu as pltpu
from jax.experimental.pallas import tpu_sc as plsc

EPS = 1e-12


def kernel(input_ids, word_emb, pos_emb, gamma, beta):
    B, S = input_ids.shape
    V, H = word_emb.shape
    assert H == 64
    NW = 32
    seq_per_w = B // NW
    S0 = 128
    S1 = S - S0
    HP = 128
    wemb128 = jnp.pad(word_emb, ((0, 0), (0, HP - H)))

    mesh = plsc.VectorSubcoreMesh(core_axis_name="c", subcore_axis_name="s")

    @functools.partial(
        pl.kernel,
        out_type=jax.ShapeDtypeStruct((B, S, H), jnp.float32),
        mesh=mesh,
        scratch_types=[
            pltpu.VMEM((2, S), jnp.int32),
            pltpu.VMEM((2, S, HP), jnp.float32),
            pltpu.VMEM((2, S, H), jnp.float32),
            pltpu.VMEM((S, H), jnp.float32),
            pltpu.VMEM((H,), jnp.float32),
            pltpu.VMEM((H,), jnp.float32),
            pltpu.SemaphoreType.DMA,
            pltpu.SemaphoreType.DMA,
            pltpu.SemaphoreType.DMA,
            pltpu.SemaphoreType.DMA,
        ],
        compiler_params=pltpu.CompilerParams(
            needs_layout_passes=False),
    )
    def emb_ln(ids_hbm, wemb_hbm, pemb_hbm, g_hbm, b_hbm, out_hbm,
               idx_v, rows_v, outb_v, pos_v, g_v, b_v,
               gsem0, gsem1, osem0, osem1):
        cid = lax.axis_index("c")
        sid = lax.axis_index("s")
        wid = sid * 2 + cid
        seq0 = wid * seq_per_w
        gsem = [gsem0, gsem1]
        osem = [osem0, osem1]

        pltpu.sync_copy(pemb_hbm.at[pl.ds(0, S)], pos_v)
        pltpu.sync_copy(g_hbm, g_v)
        pltpu.sync_copy(b_hbm, b_v)
        g = [g_v[pl.ds(16 * j, 16)] for j in range(4)]
        bt = [b_v[pl.ds(16 * j, 16)] for j in range(4)]
        fifteen = jnp.full((16,), 15, dtype=jnp.int32)

        def start_gather(seq, b):
            pltpu.sync_copy(ids_hbm.at[seq], idx_v.at[b])
            pltpu.async_copy(
                wemb_hbm.at[idx_v.at[b, pl.ds(0, S0)]],
                rows_v.at[b, pl.ds(0, S0)], gsem[b])
            pltpu.async_copy(
                wemb_hbm.at[idx_v.at[b, pl.ds(S0, S1)]],
                rows_v.at[b, pl.ds(S0, S1)], gsem[b])

        def wait_gather(seq, b):
            pltpu.make_async_copy(
                wemb_hbm.at[idx_v.at[b, pl.ds(0, S0)]],
                rows_v.at[b, pl.ds(0, S0)], gsem[b]).wait()
            pltpu.make_async_copy(
                wemb_hbm.at[idx_v.at[b, pl.ds(S0, S1)]],
                rows_v.at[b, pl.ds(S0, S1)], gsem[b]).wait()

        def drain_out(seq, b):
            pltpu.make_async_copy(
                outb_v.at[b], out_hbm.at[seq], osem[b]).wait()

        start_gather(seq0, 0)

        def pair_body(i2, carry):
            for b in range(2):
                i = i2 * 2 + b
                seq = seq0 + i
                wait_gather(seq, b)

                @pl.when(i + 1 < seq_per_w)
                def _():
                    start_gather(seq + 1, 1 - b)

                @pl.when(i >= 2)
                def _():
                    drain_out(seq - 2, b)

                @plsc.parallel_loop(0, S, unroll=4)
                def tok(t):
                    x = [rows_v[b, t, pl.ds(16 * j, 16)]
                         + pos_v[t, pl.ds(16 * j, 16)] for j in range(4)]
                    s = (x[0] + x[1]) + (x[2] + x[3])
                    q = ((x[0] * x[0] + x[1] * x[1])
                         + (x[2] * x[2] + x[3] * x[3]))
                    tot = jnp.take_along_axis(
                        plsc.cumsum(s), fifteen, axis=0,
                        mode="promise_in_bounds")
                    tot2 = jnp.take_along_axis(
                        plsc.cumsum(q), fifteen, axis=0,
                        mode="promise_in_bounds")
                    mean = tot * (1.0 / 64.0)
                    var = tot2 * (1.0 / 64.0) - mean * mean
                    v = jnp.maximum(var, 0.0) + EPS
                    iv = plsc.bitcast(v, jnp.int32)
                    y = plsc.bitcast(jnp.int32(0x5F3759DF) - (iv >> 1),
                                     jnp.float32)
                    h = 0.5 * v
                    y = y * (1.5 - h * y * y)
                    y = y * (1.5 - h * y * y)
                    y = y * (1.5 - h * y * y)
                    for j in range(4):
                        outb_v[b, t, pl.ds(16 * j, 16)] = (
                            (x[j] - mean) * y * g[j] + bt[j])

                pltpu.async_copy(outb_v.at[b], out_hbm.at[seq], osem[b])
            return carry

        lax.fori_loop(0, seq_per_w // 2, pair_body, 0)
        drain_out(seq0 + seq_per_w - 2, 0)
        drain_out(seq0 + seq_per_w - 1, 1)

    return emb_ln(input_ids, wemb128, pos_emb, gamma, beta)
